# Initial kernel scaffold; baseline (speedup 1.0000x reference)
import functools

import jax
import jax.numpy as jnp
from jax import lax
from jax.experimental import pallas as pl
from jax.experimental.pallas import tpu as pltpu
from jax.experimental.pallas import tpu_sc as plsc

N = 10000
E = 320000
D = 128

NC = 2
NS = 16
NW = NC * NS
EPT = E // NW
BLK = 100
NBLK = EPT // BLK
NROW16 = N // 16
RPT = N // NS
RCHUNK = 125
NRCHUNK = RPT // RCHUNK

_mesh = plsc.VectorSubcoreMesh(core_axis_name="c", subcore_axis_name="s")


@functools.partial(
    pl.kernel,
    out_type=jax.ShapeDtypeStruct((NC, NROW16, 16), jnp.float32),
    mesh=_mesh,
    scratch_types=[
        pltpu.VMEM((EPT,), jnp.int32),
        pltpu.VMEM((NROW16, 16), jnp.float32),
        pltpu.VMEM((NRCHUNK, RCHUNK), jnp.int32),
        pltpu.VMEM_SHARED((NROW16, 16), jnp.float32),
    ],
)
def _deg_kernel(dst_hbm, rowidx_hbm, deg_hbm, dstv, degloc, rowv, shdeg):
    c = lax.axis_index("c")
    s = lax.axis_index("s")
    chunk = c * NS + s
    zero16 = jnp.zeros((16,), jnp.float32)

    def _zero(i, _):
        degloc[i, :] = zero16
        return 0

    lax.fori_loop(0, NROW16, _zero, 0)

    @pl.when(s == 0)
    def _():
        pltpu.sync_copy(degloc, shdeg)

    pltpu.sync_copy(dst_hbm.at[chunk], dstv)
    pltpu.sync_copy(rowidx_hbm, rowv)
    plsc.subcore_barrier()

    ones16 = jnp.ones((16,), jnp.float32)

    def _hist(i, _):
        idx = dstv[pl.ds(i * 16, 16)]
        row = lax.shift_right_logical(idx, 4)
        col = lax.bitwise_and(idx, 15)
        plsc.addupdate_scatter(degloc, [row, col], ones16)
        return 0

    lax.fori_loop(0, EPT // 16, _hist, 0)

    for q in range(NRCHUNK):
        pltpu.sync_copy(degloc.at[pl.ds(q * RCHUNK, RCHUNK)],
                        shdeg.at[rowv.at[q]], add=True)
    plsc.subcore_barrier()

    @pl.when(s == 0)
    def _():
        pltpu.sync_copy(shdeg, deg_hbm.at[c])


@functools.partial(
    pl.kernel,
    out_type=jax.ShapeDtypeStruct((NC, N, D), jnp.float32),
    mesh=_mesh,
    scratch_types=[
        pltpu.VMEM((NBLK, BLK), jnp.int32),
        pltpu.VMEM((NBLK, BLK), jnp.int32),
        pltpu.VMEM((BLK, D), jnp.float32),
        pltpu.VMEM((BLK, D), jnp.float32),
        pltpu.VMEM((RCHUNK, D), jnp.float32),
        pltpu.VMEM_SHARED((N, D), jnp.float32),
        pltpu.SemaphoreType.DMA,
        pltpu.SemaphoreType.DMA,
    ],
)
def _scatter_kernel(hw_hbm, src_hbm, dst_hbm, part_hbm,
                    srcv, dstv, buf_a, buf_b, zbuf, outsh, sem_a, sem_b):
    c = lax.axis_index("c")
    s = lax.axis_index("s")
    chunk = c * NS + s
    pltpu.sync_copy(src_hbm.at[chunk], srcv)
    pltpu.sync_copy(dst_hbm.at[chunk], dstv)

    zero16 = jnp.zeros((16,), jnp.float32)

    def _zero(r, _):
        for k in range(D // 16):
            zbuf[r, pl.ds(k * 16, 16)] = zero16
        return 0

    lax.fori_loop(0, RCHUNK, _zero, 0)
    base = s * RPT
    for q in range(NRCHUNK):
        pltpu.sync_copy(zbuf, outsh.at[pl.ds(base + q * RCHUNK, RCHUNK)])
    plsc.subcore_barrier()

    pltpu.async_copy(hw_hbm.at[srcv.at[0]], buf_a, sem_a)

    def _body(jj, _):
        j0 = 2 * jj
        pltpu.async_copy(hw_hbm.at[srcv.at[j0 + 1]], buf_b, sem_b)
        pltpu.make_async_copy(hw_hbm.at[srcv.at[j0]], buf_a, sem_a).wait()
        pltpu.sync_copy(buf_a, outsh.at[dstv.at[j0]], add=True)

        @pl.when(jj < NBLK // 2 - 1)
        def _():
            pltpu.async_copy(hw_hbm.at[srcv.at[j0 + 2]], buf_a, sem_a)

        pltpu.make_async_copy(hw_hbm.at[srcv.at[j0 + 1]], buf_b, sem_b).wait()
        pltpu.sync_copy(buf_b, outsh.at[dstv.at[j0 + 1]], add=True)
        return 0

    lax.fori_loop(0, NBLK // 2, _body, 0)
    plsc.subcore_barrier()
    pltpu.sync_copy(outsh.at[pl.ds(base, RPT)],
                    part_hbm.at[c, pl.ds(base, RPT)])


_GB = 2000
_GRID = N // _GB


def _mm_body(h_ref, w_ref, o_ref):
    o_ref[...] = lax.dot_general(
        h_ref[...], w_ref[...], (((1,), (1,)), ((), ())),
        preferred_element_type=jnp.float32)


def _matmul(h, w):
    return pl.pallas_call(
        _mm_body,
        grid=(_GRID,),
        in_specs=[pl.BlockSpec((_GB, D), lambda i: (i, 0)),
                  pl.BlockSpec((D, D), lambda i: (0, 0))],
        out_specs=pl.BlockSpec((_GB, D), lambda i: (i, 0)),
        out_shape=jax.ShapeDtypeStruct((N, D), jnp.float32),
    )(h, w)


def _dis_body(degp_ref, o_ref):
    deg = degp_ref[0] + degp_ref[1] + 1.0
    o_ref[...] = lax.rsqrt(deg)


def _dis(degp):
    return pl.pallas_call(
        _dis_body,
        out_shape=jax.ShapeDtypeStruct((NROW16, 16), jnp.float32),
    )(degp)


def _scale_body(hw_ref, dis_ref, o_ref):
    o_ref[...] = hw_ref[...] * dis_ref[...]


def _scale(hw, dis):
    return pl.pallas_call(
        _scale_body,
        grid=(_GRID,),
        in_specs=[pl.BlockSpec((_GB, D), lambda i: (i, 0)),
                  pl.BlockSpec((_GB, 1), lambda i: (i, 0))],
        out_specs=pl.BlockSpec((_GB, D), lambda i: (i, 0)),
        out_shape=jax.ShapeDtypeStruct((N, D), jnp.float32),
    )(hw, dis)


def _mid_body(p0_ref, p1_ref, hwp_ref, dis_ref, b_ref, w_ref, o_ref):
    h1 = dis_ref[...] * (p0_ref[...] + p1_ref[...] + hwp_ref[...]) + b_ref[...]
    o_ref[...] = lax.dot_general(
        h1, w_ref[...], (((1,), (1,)), ((), ())),
        preferred_element_type=jnp.float32) * dis_ref[...]


def _mid(p0, p1, hwp, dis, b2, w):
    return pl.pallas_call(
        _mid_body,
        grid=(_GRID,),
        in_specs=[pl.BlockSpec((_GB, D), lambda i: (i, 0)),
                  pl.BlockSpec((_GB, D), lambda i: (i, 0)),
                  pl.BlockSpec((_GB, D), lambda i: (i, 0)),
                  pl.BlockSpec((_GB, 1), lambda i: (i, 0)),
                  pl.BlockSpec((1, D), lambda i: (0, 0)),
                  pl.BlockSpec((D, D), lambda i: (0, 0))],
        out_specs=pl.BlockSpec((_GB, D), lambda i: (i, 0)),
        out_shape=jax.ShapeDtypeStruct((N, D), jnp.float32),
    )(p0, p1, hwp, dis, b2, w)


def _fin_body(p0_ref, p1_ref, hwp_ref, dis_ref, b_ref, o_ref):
    o_ref[...] = dis_ref[...] * (p0_ref[...] + p1_ref[...] + hwp_ref[...]) \
        + b_ref[...]


def _fin(p0, p1, hwp, dis, b2):
    return pl.pallas_call(
        _fin_body,
        grid=(_GRID,),
        in_specs=[pl.BlockSpec((_GB, D), lambda i: (i, 0)),
                  pl.BlockSpec((_GB, D), lambda i: (i, 0)),
                  pl.BlockSpec((_GB, D), lambda i: (i, 0)),
                  pl.BlockSpec((_GB, 1), lambda i: (i, 0)),
                  pl.BlockSpec((1, D), lambda i: (0, 0))],
        out_specs=pl.BlockSpec((_GB, D), lambda i: (i, 0)),
        out_shape=jax.ShapeDtypeStruct((N, D), jnp.float32),
    )(p0, p1, hwp, dis, b2)


def kernel(in_feat, g, W, b):
    src = g[0].reshape(NW, NBLK, BLK)
    dst = g[1].reshape(NW, NBLK, BLK)
    dstf = g[1].reshape(NW, EPT)
    rowidx = jnp.arange(NROW16, dtype=jnp.int32).reshape(NRCHUNK, RCHUNK)

    degp = _deg_kernel(dstf, rowidx)
    hw1 = _matmul(in_feat, W)
    dis = _dis(degp).reshape(N, 1)
    hw1p = _scale(hw1, dis)
    b2 = b.reshape(1, D)

    part1 = _scatter_kernel(hw1p, src, dst)
    hw2p = _mid(part1[0], part1[1], hw1p, dis, b2, W)
    part2 = _scatter_kernel(hw2p, src, dst)
    return _fin(part2[0], part2[1], hw2p, dis, b2)


# speedup vs baseline: 13.8991x; 13.8991x over previous
import functools

import jax
import jax.numpy as jnp
from jax import lax
from jax.experimental import pallas as pl
from jax.experimental.pallas import tpu as pltpu
from jax.experimental.pallas import tpu_sc as plsc

N = 10000
E = 320000
D = 128

NC = 2
NS = 16
NW = NC * NS
EPT = E // NW
BLK = 100
NBLK = EPT // BLK
NROW16 = N // 16
RPT = N // NS
CPT = 632
ZR1 = 320
ZR2 = CPT - ZR1

_mesh = plsc.VectorSubcoreMesh(core_axis_name="c", subcore_axis_name="s")


NP = 10240


@functools.partial(
    pl.kernel,
    out_type=jax.ShapeDtypeStruct((NW, NP), jnp.float32),
    mesh=_mesh,
    scratch_types=[
        pltpu.VMEM((EPT,), jnp.int32),
        pltpu.VMEM((NP,), jnp.float32),
    ],
    compiler_params=pltpu.CompilerParams(needs_layout_passes=False),
)
def _deg_kernel(dst_hbm, deg_hbm, dstv, degloc):
    c = lax.axis_index("c")
    s = lax.axis_index("s")
    chunk = c * NS + s
    pltpu.sync_copy(dst_hbm.at[chunk], dstv)
    zero16 = jnp.zeros((16,), jnp.float32)

    def _zero(i, _):
        degloc[pl.ds(i * 16, 16)] = zero16
        return 0

    lax.fori_loop(0, NP // 16, _zero, 0)

    ones16 = jnp.ones((16,), jnp.float32)

    def _hist(i, _):
        idx = dstv[pl.ds(i * 16, 16)]
        plsc.addupdate_scatter(degloc, [idx], ones16)
        return 0

    lax.fori_loop(0, EPT // 16, _hist, 0)
    pltpu.sync_copy(degloc, deg_hbm.at[chunk])


HALF = N // NC
ACCR = 5120
SBLK = 80
SNB = 250
RPT2 = ACCR // NS


@functools.partial(
    pl.kernel,
    out_type=jax.ShapeDtypeStruct((NC, ACCR, D), jnp.float32),
    mesh=_mesh,
    scratch_types=[
        pltpu.VMEM((SNB, SBLK), jnp.int32),
        pltpu.VMEM((SNB, SBLK), jnp.int32),
        pltpu.VMEM((SBLK, D), jnp.float32),
        pltpu.VMEM((SBLK, D), jnp.float32),
        pltpu.VMEM_SHARED((ACCR, D), jnp.float32),
        pltpu.SemaphoreType.DMA,
        pltpu.SemaphoreType.DMA,
    ],
)
def _scatter_kernel(hw_hbm, src_hbm, dst_hbm, part_hbm,
                    srcv, dstv, buf_a, buf_b, outsh, sem_a, sem_b):
    c = lax.axis_index("c")
    s = lax.axis_index("s")
    pltpu.sync_copy(src_hbm.at[s], srcv)
    pltpu.sync_copy(dst_hbm.at[s], dstv)

    lo = c * HALF

    def _remap(r, _):
        for k in range(SBLK // 16):
            d16 = dstv[r, pl.ds(k * 16, 16)]
            local = d16 - lo
            ok = (local >= 0) & (local < HALF)
            dstv[r, pl.ds(k * 16, 16)] = jnp.where(ok, local, HALF)
        return 0

    lax.fori_loop(0, SNB, _remap, 0)

    zero16 = jnp.zeros((16,), jnp.float32)

    def _zero(r, _):
        for k in range(D // 16):
            buf_a[r, pl.ds(k * 16, 16)] = zero16
        return 0

    lax.fori_loop(0, SBLK, _zero, 0)
    base = s * RPT2
    for q in range(RPT2 // SBLK):
        pltpu.sync_copy(buf_a, outsh.at[pl.ds(base + q * SBLK, SBLK)])
    plsc.subcore_barrier()

    def _body(jj, _):
        j0 = 2 * jj
        ha = pltpu.async_copy(hw_hbm.at[srcv.at[j0]], buf_a, sem_a)
        hb = pltpu.async_copy(hw_hbm.at[srcv.at[j0 + 1]], buf_b, sem_b)
        ha.wait()
        pltpu.sync_copy(buf_a, outsh.at[dstv.at[j0]], add=True)
        hb.wait()
        pltpu.sync_copy(buf_b, outsh.at[dstv.at[j0 + 1]], add=True)
        return 0

    lax.fori_loop(0, SNB // 2, _body, 0)
    plsc.subcore_barrier()
    pltpu.sync_copy(outsh.at[pl.ds(base, RPT2)],
                    part_hbm.at[c, pl.ds(base, RPT2)])


_GB = 1000
_GRID = N // _GB
_HB = HALF // _GB


def _part_spec():
    return pl.BlockSpec((1, _GB, D), lambda i: (i // _HB, i % _HB, 0))


def _mm_body(h_ref, w_ref, o_ref):
    o_ref[...] = lax.dot_general(
        h_ref[...], w_ref[...], (((1,), (1,)), ((), ())),
        preferred_element_type=jnp.float32)


def _matmul(h, w):
    return pl.pallas_call(
        _mm_body,
        grid=(_GRID,),
        in_specs=[pl.BlockSpec((_GB, D), lambda i: (i, 0)),
                  pl.BlockSpec((D, D), lambda i: (0, 0))],
        out_specs=pl.BlockSpec((_GB, D), lambda i: (i, 0)),
        out_shape=jax.ShapeDtypeStruct((N, D), jnp.float32),
    )(h, w)


def _dis_body(degp_ref, o_ref):
    deg = jnp.sum(degp_ref[...], axis=0)
    o_ref[...] = lax.rsqrt(deg + 1.0)


def _dis(degp):
    return pl.pallas_call(
        _dis_body,
        out_shape=jax.ShapeDtypeStruct((NP // 128, 128), jnp.float32),
    )(degp)


def _scale_body(hw_ref, dis_ref, o_ref):
    o_ref[...] = hw_ref[...] * dis_ref[...]


def _scale(hw, dis):
    return pl.pallas_call(
        _scale_body,
        grid=(_GRID,),
        in_specs=[pl.BlockSpec((_GB, D), lambda i: (i, 0)),
                  pl.BlockSpec((_GB, 1), lambda i: (i, 0))],
        out_specs=pl.BlockSpec((_GB, D), lambda i: (i, 0)),
        out_shape=jax.ShapeDtypeStruct((N, D), jnp.float32),
    )(hw, dis)


def _mid_body(p_ref, hwp_ref, dis_ref, b_ref, w_ref, o_ref):
    h1 = dis_ref[...] * (p_ref[0] + hwp_ref[...]) + b_ref[...]
    o_ref[...] = lax.dot_general(
        h1, w_ref[...], (((1,), (1,)), ((), ())),
        preferred_element_type=jnp.float32) * dis_ref[...]


def _mid(part, hwp, dis, b2, w):
    return pl.pallas_call(
        _mid_body,
        grid=(_GRID,),
        in_specs=[_part_spec(),
                  pl.BlockSpec((_GB, D), lambda i: (i, 0)),
                  pl.BlockSpec((_GB, 1), lambda i: (i, 0)),
                  pl.BlockSpec((1, D), lambda i: (0, 0)),
                  pl.BlockSpec((D, D), lambda i: (0, 0))],
        out_specs=pl.BlockSpec((_GB, D), lambda i: (i, 0)),
        out_shape=jax.ShapeDtypeStruct((N, D), jnp.float32),
    )(part, hwp, dis, b2, w)


def _fin_body(p_ref, hwp_ref, dis_ref, b_ref, o_ref):
    o_ref[...] = dis_ref[...] * (p_ref[0] + hwp_ref[...]) + b_ref[...]


def _fin(part, hwp, dis, b2):
    return pl.pallas_call(
        _fin_body,
        grid=(_GRID,),
        in_specs=[_part_spec(),
                  pl.BlockSpec((_GB, D), lambda i: (i, 0)),
                  pl.BlockSpec((_GB, 1), lambda i: (i, 0)),
                  pl.BlockSpec((1, D), lambda i: (0, 0))],
        out_specs=pl.BlockSpec((_GB, D), lambda i: (i, 0)),
        out_shape=jax.ShapeDtypeStruct((N, D), jnp.float32),
    )(part, hwp, dis, b2)


def kernel(in_feat, g, W, b):
    src = g[0].reshape(NS, SNB, SBLK)
    dst = g[1].reshape(NS, SNB, SBLK)
    dstf = g[1].reshape(NW, EPT)

    degp = _deg_kernel(dstf)
    hw1 = _matmul(in_feat, W)
    dis = _dis(degp.reshape(NW, NP // 128, 128)).reshape(NP, 1)[:N]
    hw1p = _scale(hw1, dis)
    b2 = b.reshape(1, D)

    part1 = _scatter_kernel(hw1p, src, dst)
    hw2p = _mid(part1, hw1p, dis, b2, W)
    part2 = _scatter_kernel(hw2p, src, dst)
    return _fin(part2, hw2p, dis, b2)


# speedup vs baseline: 16.9122x; 1.2168x over previous
import functools

import jax
import jax.numpy as jnp
from jax import lax
from jax.experimental import pallas as pl
from jax.experimental.pallas import tpu as pltpu
from jax.experimental.pallas import tpu_sc as plsc

N = 10000
E = 320000
D = 128

NC = 2
NS = 16
NW = NC * NS
EPT = E // NW
BLK = 100
NBLK = EPT // BLK
NROW16 = N // 16
RPT = N // NS
CPT = 632
ZR1 = 320
ZR2 = CPT - ZR1

_mesh = plsc.VectorSubcoreMesh(core_axis_name="c", subcore_axis_name="s")


NP = 10240


@functools.partial(
    pl.kernel,
    out_type=jax.ShapeDtypeStruct((NW, NP), jnp.float32),
    mesh=_mesh,
    scratch_types=[
        pltpu.VMEM((EPT,), jnp.int32),
        pltpu.VMEM((NP,), jnp.float32),
    ],
    compiler_params=pltpu.CompilerParams(needs_layout_passes=False),
)
def _deg_kernel(dst_hbm, deg_hbm, dstv, degloc):
    c = lax.axis_index("c")
    s = lax.axis_index("s")
    chunk = c * NS + s
    pltpu.sync_copy(dst_hbm.at[chunk], dstv)
    zero16 = jnp.zeros((16,), jnp.float32)

    def _zero(i, _):
        degloc[pl.ds(i * 16, 16)] = zero16
        return 0

    lax.fori_loop(0, NP // 16, _zero, 0)

    ones16 = jnp.ones((16,), jnp.float32)

    def _hist(i, _):
        idx = dstv[pl.ds(i * 16, 16)]
        plsc.addupdate_scatter(degloc, [idx], ones16)
        return 0

    lax.fori_loop(0, EPT // 16, _hist, 0)
    pltpu.sync_copy(degloc, deg_hbm.at[chunk])


HALF = N // NC
ACCR = 5120
SBLK = 80
SNB = 250
RPT2 = ACCR // NS
NSLOT = 2
NGRP = SNB // NSLOT
NTAIL = SNB - NGRP * NSLOT


@functools.partial(
    pl.kernel,
    out_type=jax.ShapeDtypeStruct((NC, ACCR, D), jnp.float32),
    mesh=_mesh,
    scratch_types=[
        pltpu.VMEM((SNB, SBLK), jnp.int32),
        pltpu.VMEM((SNB, SBLK), jnp.int32),
        [pltpu.VMEM((SBLK, D), jnp.float32) for _ in range(NSLOT)],
        pltpu.VMEM_SHARED((ACCR, D), jnp.float32),
        [pltpu.SemaphoreType.DMA for _ in range(NSLOT)],
    ],
)
def _scatter_kernel(hw_hbm, src_hbm, dst_hbm, part_hbm,
                    srcv, dstv, bufs, outsh, sems):
    c = lax.axis_index("c")
    s = lax.axis_index("s")
    pltpu.sync_copy(src_hbm.at[s], srcv)
    pltpu.sync_copy(dst_hbm.at[s], dstv)

    lo = c * HALF

    def _remap(r, _):
        for k in range(SBLK // 16):
            d16 = dstv[r, pl.ds(k * 16, 16)]
            local = d16 - lo
            ok = (local >= 0) & (local < HALF)
            dstv[r, pl.ds(k * 16, 16)] = jnp.where(ok, local, HALF)
        return 0

    lax.fori_loop(0, SNB, _remap, 0)

    zero16 = jnp.zeros((16,), jnp.float32)

    def _zero(r, _):
        for k in range(D // 16):
            bufs[0][r, pl.ds(k * 16, 16)] = zero16
        return 0

    lax.fori_loop(0, SBLK, _zero, 0)
    base = s * RPT2
    for q in range(RPT2 // SBLK):
        pltpu.sync_copy(bufs[0], outsh.at[pl.ds(base + q * SBLK, SBLK)])
    plsc.subcore_barrier()

    def _issue(j, k):
        pltpu.async_copy(hw_hbm.at[srcv.at[j]], bufs[k], sems[k])

    def _drain(j, k):
        pltpu.make_async_copy(hw_hbm.at[srcv.at[j]], bufs[k], sems[k]).wait()

    for k in range(NSLOT):
        _issue(k, k)

    def _body(jj, _):
        j0 = jj * NSLOT
        for k in range(NSLOT):
            j = j0 + k
            _drain(j, k)
            pltpu.sync_copy(bufs[k], outsh.at[dstv.at[j]], add=True)

            @pl.when(j + NSLOT < SNB)
            def _(j=j, k=k):
                _issue(j + NSLOT, k)
        return 0

    lax.fori_loop(0, NGRP, _body, 0)
    for t in range(NTAIL):
        j = NGRP * NSLOT + t
        _drain(j, t)
        pltpu.sync_copy(bufs[t], outsh.at[dstv.at[j]], add=True)
    plsc.subcore_barrier()
    pltpu.sync_copy(outsh.at[pl.ds(base, RPT2)],
                    part_hbm.at[c, pl.ds(base, RPT2)])


_GB = 1000
_GRID = N // _GB
_HB = HALF // _GB


def _part_spec():
    return pl.BlockSpec((1, _GB, D), lambda i: (i // _HB, i % _HB, 0))


def _mm_body(h_ref, w_ref, o_ref):
    o_ref[...] = lax.dot_general(
        h_ref[...], w_ref[...], (((1,), (1,)), ((), ())),
        preferred_element_type=jnp.float32)


def _matmul(h, w):
    return pl.pallas_call(
        _mm_body,
        grid=(_GRID,),
        in_specs=[pl.BlockSpec((_GB, D), lambda i: (i, 0)),
                  pl.BlockSpec((D, D), lambda i: (0, 0))],
        out_specs=pl.BlockSpec((_GB, D), lambda i: (i, 0)),
        out_shape=jax.ShapeDtypeStruct((N, D), jnp.float32),
    )(h, w)


def _dis_body(degp_ref, o_ref):
    deg = jnp.sum(degp_ref[...], axis=0)
    o_ref[...] = lax.rsqrt(deg + 1.0)


def _dis(degp):
    return pl.pallas_call(
        _dis_body,
        out_shape=jax.ShapeDtypeStruct((NP // 128, 128), jnp.float32),
    )(degp)


def _scale_body(hw_ref, dis_ref, o_ref):
    o_ref[...] = hw_ref[...] * dis_ref[...]


def _scale(hw, dis):
    return pl.pallas_call(
        _scale_body,
        grid=(_GRID,),
        in_specs=[pl.BlockSpec((_GB, D), lambda i: (i, 0)),
                  pl.BlockSpec((_GB, 1), lambda i: (i, 0))],
        out_specs=pl.BlockSpec((_GB, D), lambda i: (i, 0)),
        out_shape=jax.ShapeDtypeStruct((N, D), jnp.float32),
    )(hw, dis)


def _mid_body(p_ref, hwp_ref, dis_ref, b_ref, w_ref, o_ref):
    h1 = dis_ref[...] * (p_ref[0] + hwp_ref[...]) + b_ref[...]
    o_ref[...] = lax.dot_general(
        h1, w_ref[...], (((1,), (1,)), ((), ())),
        preferred_element_type=jnp.float32) * dis_ref[...]


def _mid(part, hwp, dis, b2, w):
    return pl.pallas_call(
        _mid_body,
        grid=(_GRID,),
        in_specs=[_part_spec(),
                  pl.BlockSpec((_GB, D), lambda i: (i, 0)),
                  pl.BlockSpec((_GB, 1), lambda i: (i, 0)),
                  pl.BlockSpec((1, D), lambda i: (0, 0)),
                  pl.BlockSpec((D, D), lambda i: (0, 0))],
        out_specs=pl.BlockSpec((_GB, D), lambda i: (i, 0)),
        out_shape=jax.ShapeDtypeStruct((N, D), jnp.float32),
    )(part, hwp, dis, b2, w)


def _fin_body(p_ref, hwp_ref, dis_ref, b_ref, o_ref):
    o_ref[...] = dis_ref[...] * (p_ref[0] + hwp_ref[...]) + b_ref[...]


def _fin(part, hwp, dis, b2):
    return pl.pallas_call(
        _fin_body,
        grid=(_GRID,),
        in_specs=[_part_spec(),
                  pl.BlockSpec((_GB, D), lambda i: (i, 0)),
                  pl.BlockSpec((_GB, 1), lambda i: (i, 0)),
                  pl.BlockSpec((1, D), lambda i: (0, 0))],
        out_specs=pl.BlockSpec((_GB, D), lambda i: (i, 0)),
        out_shape=jax.ShapeDtypeStruct((N, D), jnp.float32),
    )(part, hwp, dis, b2)


def kernel(in_feat, g, W, b):
    src = g[0].reshape(NS, SNB, SBLK)
    dst = g[1].reshape(NS, SNB, SBLK)
    dstf = g[1].reshape(NW, EPT)

    degp = _deg_kernel(dstf)
    hw1 = _matmul(in_feat, W)
    dis = _dis(degp.reshape(NW, NP // 128, 128)).reshape(NP, 1)[:N]
    hw1p = _scale(hw1, dis)
    b2 = b.reshape(1, D)

    part1 = _scatter_kernel(hw1p, src, dst)
    hw2p = _mid(part1, hw1p, dis, b2, W)
    part2 = _scatter_kernel(hw2p, src, dst)
    return _fin(part2, hw2p, dis, b2)


# speedup vs baseline: 18.7172x; 1.1067x over previous
import functools

import jax
import jax.numpy as jnp
from jax import lax
from jax.experimental import pallas as pl
from jax.experimental.pallas import tpu as pltpu
from jax.experimental.pallas import tpu_sc as plsc

N = 10000
E = 320000
D = 128

NC = 2
NS = 16
NW = NC * NS
EPT = E // NW
BLK = 100
NBLK = EPT // BLK
NROW16 = N // 16
RPT = N // NS
CPT = 632
ZR1 = 320
ZR2 = CPT - ZR1

_mesh = plsc.VectorSubcoreMesh(core_axis_name="c", subcore_axis_name="s")


NP = 10240


@functools.partial(
    pl.kernel,
    out_type=jax.ShapeDtypeStruct((NW, NP), jnp.float32),
    mesh=_mesh,
    scratch_types=[
        pltpu.VMEM((EPT,), jnp.int32),
        pltpu.VMEM((NP,), jnp.float32),
    ],
    compiler_params=pltpu.CompilerParams(needs_layout_passes=False),
)
def _deg_kernel(dst_hbm, deg_hbm, dstv, degloc):
    c = lax.axis_index("c")
    s = lax.axis_index("s")
    chunk = c * NS + s
    pltpu.sync_copy(dst_hbm.at[chunk], dstv)
    zero16 = jnp.zeros((16,), jnp.float32)

    def _zero(i, _):
        degloc[pl.ds(i * 16, 16)] = zero16
        return 0

    lax.fori_loop(0, NP // 16, _zero, 0)

    ones16 = jnp.ones((16,), jnp.float32)

    def _hist(i, _):
        idx = dstv[pl.ds(i * 16, 16)]
        plsc.addupdate_scatter(degloc, [idx], ones16)
        return 0

    lax.fori_loop(0, EPT // 16, _hist, 0)
    pltpu.sync_copy(degloc, deg_hbm.at[chunk])


HALF = N // NC
ACCR = 5120
SBLK = 80
EPC = E // NS
CAP = EPC + 160
RPT2 = ACCR // NS
NSLOT = 2


@functools.partial(
    pl.kernel,
    out_type=jax.ShapeDtypeStruct((NC, ACCR, D), jnp.float32),
    mesh=_mesh,
    scratch_types=[
        pltpu.VMEM((CAP,), jnp.int32),
        pltpu.VMEM((CAP,), jnp.int32),
        [pltpu.VMEM((SBLK, D), jnp.float32) for _ in range(NSLOT)],
        pltpu.VMEM_SHARED((ACCR, D), jnp.float32),
        [pltpu.SemaphoreType.DMA for _ in range(NSLOT)],
    ],
    compiler_params=pltpu.CompilerParams(needs_layout_passes=False),
)
def _scatter_kernel(hw_hbm, src_hbm, dst_hbm, part_hbm,
                    sf, df, bufs, outsh, sems):
    c = lax.axis_index("c")
    s = lax.axis_index("s")
    pltpu.sync_copy(src_hbm.at[s], sf)
    pltpu.sync_copy(dst_hbm.at[s], df)

    lo = c * HALF

    def _compact(i, cnt):
        s16 = sf[pl.ds(i * 16, 16)]
        local = df[pl.ds(i * 16, 16)] - lo
        ok = (local >= 0) & (local < HALF)
        plsc.store_compressed(sf.at[pl.ds(cnt, 16)], s16, mask=ok)
        plsc.store_compressed(df.at[pl.ds(cnt, 16)], local, mask=ok)
        pc = jnp.max(plsc.all_reduce_population_count(ok))
        return cnt + pc

    cnt = lax.fori_loop(0, CAP // 16, _compact, jnp.int32(0))

    nbu = (cnt + SBLK - 1) // SBLK
    nb = (nbu + NSLOT - 1) // NSLOT * NSLOT
    zero16i = jnp.zeros((16,), jnp.int32)
    trash16 = jnp.full((16,), HALF, jnp.int32)
    npad = (nb * SBLK - cnt + 15) // 16

    def _pad(t, _):
        off = cnt + t * 16
        sf[pl.ds(off, 16)] = zero16i
        df[pl.ds(off, 16)] = trash16
        return 0

    lax.fori_loop(0, npad, _pad, 0)

    zero16 = jnp.zeros((16,), jnp.float32)

    def _zero(r, _):
        for k in range(D // 16):
            bufs[0][r, pl.ds(k * 16, 16)] = zero16
        return 0

    lax.fori_loop(0, SBLK, _zero, 0)
    base = s * RPT2
    for q in range(RPT2 // SBLK):
        pltpu.sync_copy(bufs[0], outsh.at[pl.ds(base + q * SBLK, SBLK)])
    plsc.subcore_barrier()

    def _issue(j, k):
        pltpu.async_copy(hw_hbm.at[sf.at[pl.ds(j * SBLK, SBLK)]],
                         bufs[k], sems[k])

    def _drain(j, k):
        pltpu.make_async_copy(hw_hbm.at[sf.at[pl.ds(j * SBLK, SBLK)]],
                              bufs[k], sems[k]).wait()

    for k in range(NSLOT):
        @pl.when(k < nb)
        def _(k=k):
            _issue(k, k)

    def _body(jj, _):
        j0 = jj * NSLOT
        for k in range(NSLOT):
            j = j0 + k
            _drain(j, k)
            pltpu.sync_copy(bufs[k], outsh.at[df.at[pl.ds(j * SBLK, SBLK)]],
                            add=True)

            @pl.when(j + NSLOT < nb)
            def _(j=j, k=k):
                _issue(j + NSLOT, k)
        return 0

    lax.fori_loop(0, nb // NSLOT, _body, 0)
    plsc.subcore_barrier()
    pltpu.sync_copy(outsh.at[pl.ds(base, RPT2)],
                    part_hbm.at[c, pl.ds(base, RPT2)])


_GB = 1000
_GRID = N // _GB
_HB = HALF // _GB


def _part_spec():
    return pl.BlockSpec((1, _GB, D), lambda i: (i // _HB, i % _HB, 0))


def _mm_body(h_ref, w_ref, o_ref):
    o_ref[...] = lax.dot_general(
        h_ref[...], w_ref[...], (((1,), (1,)), ((), ())),
        preferred_element_type=jnp.float32)


def _matmul(h, w):
    return pl.pallas_call(
        _mm_body,
        grid=(_GRID,),
        in_specs=[pl.BlockSpec((_GB, D), lambda i: (i, 0)),
                  pl.BlockSpec((D, D), lambda i: (0, 0))],
        out_specs=pl.BlockSpec((_GB, D), lambda i: (i, 0)),
        out_shape=jax.ShapeDtypeStruct((N, D), jnp.float32),
    )(h, w)


def _dis_body(degp_ref, o_ref):
    deg = jnp.sum(degp_ref[...], axis=0)
    o_ref[...] = lax.rsqrt(deg + 1.0)


def _dis(degp):
    return pl.pallas_call(
        _dis_body,
        out_shape=jax.ShapeDtypeStruct((NP // 128, 128), jnp.float32),
    )(degp)


def _scale_body(hw_ref, dis_ref, o_ref):
    o_ref[...] = hw_ref[...] * dis_ref[...]


def _scale(hw, dis):
    return pl.pallas_call(
        _scale_body,
        grid=(_GRID,),
        in_specs=[pl.BlockSpec((_GB, D), lambda i: (i, 0)),
                  pl.BlockSpec((_GB, 1), lambda i: (i, 0))],
        out_specs=pl.BlockSpec((_GB, D), lambda i: (i, 0)),
        out_shape=jax.ShapeDtypeStruct((N, D), jnp.float32),
    )(hw, dis)


def _mid_body(p_ref, hwp_ref, dis_ref, b_ref, w_ref, o_ref):
    h1 = dis_ref[...] * (p_ref[0] + hwp_ref[...]) + b_ref[...]
    o_ref[...] = lax.dot_general(
        h1, w_ref[...], (((1,), (1,)), ((), ())),
        preferred_element_type=jnp.float32) * dis_ref[...]


def _mid(part, hwp, dis, b2, w):
    return pl.pallas_call(
        _mid_body,
        grid=(_GRID,),
        in_specs=[_part_spec(),
                  pl.BlockSpec((_GB, D), lambda i: (i, 0)),
                  pl.BlockSpec((_GB, 1), lambda i: (i, 0)),
                  pl.BlockSpec((1, D), lambda i: (0, 0)),
                  pl.BlockSpec((D, D), lambda i: (0, 0))],
        out_specs=pl.BlockSpec((_GB, D), lambda i: (i, 0)),
        out_shape=jax.ShapeDtypeStruct((N, D), jnp.float32),
    )(part, hwp, dis, b2, w)


def _fin_body(p_ref, hwp_ref, dis_ref, b_ref, o_ref):
    o_ref[...] = dis_ref[...] * (p_ref[0] + hwp_ref[...]) + b_ref[...]


def _fin(part, hwp, dis, b2):
    return pl.pallas_call(
        _fin_body,
        grid=(_GRID,),
        in_specs=[_part_spec(),
                  pl.BlockSpec((_GB, D), lambda i: (i, 0)),
                  pl.BlockSpec((_GB, 1), lambda i: (i, 0)),
                  pl.BlockSpec((1, D), lambda i: (0, 0))],
        out_specs=pl.BlockSpec((_GB, D), lambda i: (i, 0)),
        out_shape=jax.ShapeDtypeStruct((N, D), jnp.float32),
    )(part, hwp, dis, b2)


def kernel(in_feat, g, W, b):
    src = jnp.pad(g[0].reshape(NS, EPC), ((0, 0), (0, CAP - EPC)))
    dst = jnp.pad(g[1].reshape(NS, EPC), ((0, 0), (0, CAP - EPC)),
                  constant_values=N)
    dstf = g[1].reshape(NW, EPT)

    degp = _deg_kernel(dstf)
    hw1 = _matmul(in_feat, W)
    dis = _dis(degp.reshape(NW, NP // 128, 128)).reshape(NP, 1)[:N]
    hw1p = _scale(hw1, dis)
    b2 = b.reshape(1, D)

    part1 = _scatter_kernel(hw1p, src, dst)
    hw2p = _mid(part1, hw1p, dis, b2, W)
    part2 = _scatter_kernel(hw2p, src, dst)
    return _fin(part2, hw2p, dis, b2)
